# baseline (speedup 1.0000x reference)
import functools

import jax
import jax.numpy as jnp
from jax import lax
from jax.experimental import pallas as pl
from jax.experimental.pallas import tpu as pltpu
from jax.experimental.pallas import tpu_sc as plsc


@functools.cache
def _make_gather(V, Ep, B, C):
    info = plsc.get_sparse_core_info()
    num_workers = info.num_cores * info.num_subcores
    b_per_w = B // num_workers
    n_chunks = b_per_w // C
    assert b_per_w % C == 0 and B % num_workers == 0

    mesh = plsc.VectorSubcoreMesh(core_axis_name="c", subcore_axis_name="s")

    @functools.partial(
        pl.kernel,
        mesh=mesh,
        out_type=jax.ShapeDtypeStruct((B, Ep), jnp.float32),
        scratch_types=[
            pltpu.VMEM((C,), jnp.int32),
            pltpu.VMEM((C, Ep), jnp.float32),
            pltpu.SemaphoreType.DMA,
        ],
    )
    def gather_kernel(table_hbm, idx_hbm, out_hbm, idx_v, rows_v, sem):
        wid = lax.axis_index("s") * info.num_cores + lax.axis_index("c")
        base = wid * b_per_w

        def body(i, carry):
            off = base + i * C
            pltpu.sync_copy(idx_hbm.at[pl.ds(off, C)], idx_v)
            pltpu.async_copy(table_hbm.at[idx_v], rows_v, sem).wait()
            pltpu.sync_copy(rows_v, out_hbm.at[pl.ds(off, C)])
            return carry

        lax.fori_loop(0, n_chunks, body, 0)

    return gather_kernel


def _head_body(emb_ref, w_ref, b_ref, out_ref):
    rb, t, vo = out_ref.shape
    acc = (
        jnp.dot(emb_ref[...], w_ref[...], preferred_element_type=jnp.float32)
        + b_ref[...]
    )
    out_ref[...] = acc.reshape(rb, t, vo)


@functools.cache
def _make_head(Bb, T, Ep, Vo, RB):
    grid = (Bb // RB,)
    return pl.pallas_call(
        _head_body,
        grid=grid,
        in_specs=[
            pl.BlockSpec((RB * T, Ep), lambda i: (i, 0)),
            pl.BlockSpec((Ep, Vo), lambda i: (0, 0)),
            pl.BlockSpec((1, Vo), lambda i: (0, 0)),
        ],
        out_specs=pl.BlockSpec((RB, T, Vo), lambda i: (i, 0, 0)),
        out_shape=jax.ShapeDtypeStruct((Bb, T, Vo), jnp.float32),
    )


def kernel(idx, tok_table, pos_table, W, b):
    Bb, T = idx.shape
    V, E = tok_table.shape
    Vo = W.shape[1]
    B = Bb * T
    Ep = 128

    tok_p = jnp.pad(tok_table, ((0, 0), (0, Ep - E)))
    W_p = jnp.pad(W, ((0, Ep - E), (0, 0)))
    flat_idx = idx.reshape(-1).astype(jnp.int32)

    emb = _make_gather(V, Ep, B, 512)(tok_p, flat_idx)
    return _make_head(Bb, T, Ep, Vo, 64)(emb, W_p, b.reshape(1, Vo))


# speedup vs baseline: 1.0257x; 1.0257x over previous
import functools

import jax
import jax.numpy as jnp
from jax import lax
from jax.experimental import pallas as pl
from jax.experimental.pallas import tpu as pltpu
from jax.experimental.pallas import tpu_sc as plsc


@functools.cache
def _make_gather(V, Ep, B, C):
    info = plsc.get_sparse_core_info()
    num_workers = info.num_cores * info.num_subcores
    b_per_w = B // num_workers
    n_chunks = b_per_w // C
    assert b_per_w % C == 0 and B % num_workers == 0

    mesh = plsc.VectorSubcoreMesh(core_axis_name="c", subcore_axis_name="s")

    @functools.partial(
        pl.kernel,
        mesh=mesh,
        out_type=jax.ShapeDtypeStruct((B, Ep), jnp.float32),
        scratch_types=[
            pltpu.VMEM((C,), jnp.int32),
            pltpu.VMEM((C, Ep), jnp.float32),
            pltpu.SemaphoreType.DMA,
        ],
    )
    def gather_kernel(table_hbm, idx_hbm, out_hbm, idx_v, rows_v, sem):
        wid = lax.axis_index("s") * info.num_cores + lax.axis_index("c")
        base = wid * b_per_w

        def body(i, carry):
            off = base + i * C
            pltpu.sync_copy(idx_hbm.at[pl.ds(off, C)], idx_v)
            pltpu.async_copy(table_hbm.at[idx_v], rows_v, sem).wait()
            pltpu.sync_copy(rows_v, out_hbm.at[pl.ds(off, C)])
            return carry

        lax.fori_loop(0, n_chunks, body, 0)

    return gather_kernel


@functools.cache
def _make_head(Bb, T, Ep, Vo, RB):
    grid_n = Bb // RB
    R = RB * T

    def body(emb_ref, w_ref, b_ref, out_hbm, acc_ref, sems):
        i = pl.program_id(0)
        slot = i % 2
        out2d = out_hbm.reshape(Bb * T, Vo)

        @pl.when(i >= 2)
        def _():
            pltpu.make_async_copy(
                acc_ref.at[slot], out2d.at[pl.ds((i - 2) * R, R)], sems.at[slot]
            ).wait()

        acc_ref[slot] = (
            jnp.dot(emb_ref[...], w_ref[...], preferred_element_type=jnp.float32)
            + b_ref[...]
        )
        pltpu.make_async_copy(
            acc_ref.at[slot], out2d.at[pl.ds(i * R, R)], sems.at[slot]
        ).start()

        @pl.when(i == grid_n - 1)
        def _():
            pltpu.make_async_copy(
                acc_ref.at[slot], out2d.at[pl.ds(i * R, R)], sems.at[slot]
            ).wait()

            @pl.when(grid_n >= 2)
            def _():
                pltpu.make_async_copy(
                    acc_ref.at[1 - slot],
                    out2d.at[pl.ds((i - 1) * R, R)],
                    sems.at[1 - slot],
                ).wait()

    return pl.pallas_call(
        body,
        grid=(grid_n,),
        in_specs=[
            pl.BlockSpec((R, Ep), lambda i: (i, 0)),
            pl.BlockSpec((Ep, Vo), lambda i: (0, 0)),
            pl.BlockSpec((1, Vo), lambda i: (0, 0)),
        ],
        out_specs=pl.BlockSpec(memory_space=pl.ANY),
        out_shape=jax.ShapeDtypeStruct((Bb, T, Vo), jnp.float32),
        scratch_shapes=[
            pltpu.VMEM((2, R, Vo), jnp.float32),
            pltpu.SemaphoreType.DMA((2,)),
        ],
    )


def kernel(idx, tok_table, pos_table, W, b):
    Bb, T = idx.shape
    V, E = tok_table.shape
    Vo = W.shape[1]
    B = Bb * T
    Ep = 128

    tok_p = jnp.pad(tok_table, ((0, 0), (0, Ep - E)))
    W_p = jnp.pad(W, ((0, Ep - E), (0, 0)))
    flat_idx = idx.reshape(-1).astype(jnp.int32)

    emb = _make_gather(V, Ep, B, 512)(tok_p, flat_idx)
    return _make_head(Bb, T, Ep, Vo, 64)(emb, W_p, b.reshape(1, Vo))


# speedup vs baseline: 1.0280x; 1.0022x over previous
import functools

import jax
import jax.numpy as jnp
from jax import lax
from jax.experimental import pallas as pl
from jax.experimental.pallas import tpu as pltpu
from jax.experimental.pallas import tpu_sc as plsc


@functools.cache
def _make_gather(V, Ep, B, C):
    info = plsc.get_sparse_core_info()
    num_workers = info.num_cores * info.num_subcores
    b_per_w = B // num_workers
    n_chunks = b_per_w // C
    assert b_per_w % C == 0 and B % num_workers == 0

    mesh = plsc.VectorSubcoreMesh(core_axis_name="c", subcore_axis_name="s")

    @functools.partial(
        pl.kernel,
        mesh=mesh,
        out_type=jax.ShapeDtypeStruct((B, Ep), jnp.float32),
        scratch_types=[
            pltpu.VMEM((C,), jnp.int32),
            pltpu.VMEM((C, Ep), jnp.float32),
            pltpu.SemaphoreType.DMA,
        ],
    )
    def gather_kernel(table_hbm, idx_hbm, out_hbm, idx_v, rows_v, sem):
        wid = lax.axis_index("s") * info.num_cores + lax.axis_index("c")
        base = wid * b_per_w

        def body(i, carry):
            off = base + i * C
            pltpu.sync_copy(idx_hbm.at[pl.ds(off, C)], idx_v)
            pltpu.async_copy(table_hbm.at[idx_v], rows_v, sem).wait()
            pltpu.sync_copy(rows_v, out_hbm.at[pl.ds(off, C)])
            return carry

        lax.fori_loop(0, n_chunks, body, 0)

    return gather_kernel


@functools.cache
def _make_head(Bb, T, Ep, Vo, RB):
    grid_n = Bb // RB
    R = RB * T

    def body(emb_ref, w_ref, b_ref, out_hbm, acc_ref, sems):
        i = pl.program_id(0)
        slot = i % 2
        @pl.when(i >= 2)
        def _():
            pltpu.make_async_copy(
                acc_ref.at[slot], out_hbm.at[pl.ds((i - 2) * RB, RB)], sems.at[slot]
            ).wait()

        acc_ref[slot] = (
            jnp.dot(emb_ref[...], w_ref[...], preferred_element_type=jnp.float32)
            + b_ref[...]
        ).reshape(RB, T, Vo)
        pltpu.make_async_copy(
            acc_ref.at[slot], out_hbm.at[pl.ds(i * RB, RB)], sems.at[slot]
        ).start()

        @pl.when(i == grid_n - 1)
        def _():
            pltpu.make_async_copy(
                acc_ref.at[slot], out_hbm.at[pl.ds(i * RB, RB)], sems.at[slot]
            ).wait()

            @pl.when(grid_n >= 2)
            def _():
                pltpu.make_async_copy(
                    acc_ref.at[1 - slot],
                    out_hbm.at[pl.ds((i - 1) * RB, RB)],
                    sems.at[1 - slot],
                ).wait()

    return pl.pallas_call(
        body,
        grid=(grid_n,),
        in_specs=[
            pl.BlockSpec((R, Ep), lambda i: (i, 0)),
            pl.BlockSpec((Ep, Vo), lambda i: (0, 0)),
            pl.BlockSpec((1, Vo), lambda i: (0, 0)),
        ],
        out_specs=pl.BlockSpec(memory_space=pl.ANY),
        out_shape=jax.ShapeDtypeStruct((Bb, T, Vo), jnp.float32),
        scratch_shapes=[
            pltpu.VMEM((2, RB, T, Vo), jnp.float32),
            pltpu.SemaphoreType.DMA((2,)),
        ],
    )


def kernel(idx, tok_table, pos_table, W, b):
    Bb, T = idx.shape
    V, E = tok_table.shape
    Vo = W.shape[1]
    B = Bb * T
    Ep = 128

    tok_p = jnp.pad(tok_table, ((0, 0), (0, Ep - E)))
    W_p = jnp.pad(W, ((0, Ep - E), (0, 0)))
    flat_idx = idx.reshape(-1).astype(jnp.int32)

    emb = _make_gather(V, Ep, B, 512)(tok_p, flat_idx)
    return _make_head(Bb, T, Ep, Vo, 64)(emb, W_p, b.reshape(1, Vo))


# speedup vs baseline: 2.7754x; 2.6998x over previous
import functools

import jax
import jax.numpy as jnp
from jax import lax
from jax.experimental import pallas as pl
from jax.experimental.pallas import tpu as pltpu
from jax.experimental.pallas import tpu_sc as plsc


@functools.cache
def _make_gather(V, Ep, T, Bb, C):
    info = plsc.get_sparse_core_info()
    num_workers = info.num_cores * info.num_subcores
    per_w = T * Bb // num_workers
    n_chunks = per_w // C
    assert per_w % C == 0 and Bb % per_w == 0
    wpt = Bb // per_w

    mesh = plsc.VectorSubcoreMesh(core_axis_name="c", subcore_axis_name="s")

    @functools.partial(
        pl.kernel,
        mesh=mesh,
        out_type=jax.ShapeDtypeStruct((T, Bb, Ep), jnp.float32),
        scratch_types=[
            pltpu.VMEM((C,), jnp.int32),
            pltpu.VMEM((C, Ep), jnp.float32),
            pltpu.SemaphoreType.DMA,
        ],
    )
    def gather_kernel(table_hbm, idx_hbm, out_hbm, idx_v, rows_v, sem):
        wid = lax.axis_index("s") * info.num_cores + lax.axis_index("c")
        base = wid * per_w
        t_id = wid // wpt
        b_base = (wid % wpt) * per_w

        def body(i, carry):
            off = base + i * C
            pltpu.sync_copy(idx_hbm.at[pl.ds(off, C)], idx_v)
            pltpu.async_copy(table_hbm.at[idx_v], rows_v, sem).wait()
            pltpu.sync_copy(
                rows_v, out_hbm.at[t_id, pl.ds(b_base + i * C, C), :]
            )
            return carry

        lax.fori_loop(0, n_chunks, body, 0)

    return gather_kernel


def _head_body(emb_ref, wt_ref, b_ref, out_ref):
    _, vc, bb = out_ref.shape
    acc = (
        lax.dot_general(
            wt_ref[...],
            emb_ref[0],
            dimension_numbers=(((1,), (1,)), ((), ())),
            preferred_element_type=jnp.float32,
        )
        + b_ref[...]
    )
    out_ref[...] = acc.reshape(1, vc, bb)


@functools.cache
def _make_head_t(T, Ep, Bb, Vo, VC, BC):
    grid = (T, Bb // BC, Vo // VC)
    return pl.pallas_call(
        _head_body,
        grid=grid,
        in_specs=[
            pl.BlockSpec((1, BC, Ep), lambda t, k, j: (t, k, 0)),
            pl.BlockSpec((VC, Ep), lambda t, k, j: (j, 0)),
            pl.BlockSpec((VC, 1), lambda t, k, j: (j, 0)),
        ],
        out_specs=pl.BlockSpec((1, VC, BC), lambda t, k, j: (t, j, k)),
        out_shape=jax.ShapeDtypeStruct((T, Vo, Bb), jnp.float32),
    )


def kernel(idx, tok_table, pos_table, W, b):
    Bb, T = idx.shape
    V, E = tok_table.shape
    Vo = W.shape[1]
    Ep = 128

    tok_p = jnp.pad(tok_table, ((0, 0), (0, Ep - E)))
    Wt_p = jnp.pad(W.T, ((0, 0), (0, Ep - E)))
    bcol = b.reshape(Vo, 1)
    idx_t = idx.T.reshape(-1).astype(jnp.int32)

    emb3 = _make_gather(V, Ep, T, Bb, 512)(tok_p, idx_t)
    out_tr = _make_head_t(T, Ep, Bb, Vo, 200, 8192)(emb3, Wt_p, bcol)
    return jnp.transpose(out_tr, (2, 0, 1))


# speedup vs baseline: 2.7758x; 1.0002x over previous
import functools

import jax
import jax.numpy as jnp
from jax import lax
from jax.experimental import pallas as pl
from jax.experimental.pallas import tpu as pltpu
from jax.experimental.pallas import tpu_sc as plsc


@functools.cache
def _make_gather(V, Ep, T, Bb, C):
    info = plsc.get_sparse_core_info()
    num_workers = info.num_cores * info.num_subcores
    per_w = T * Bb // num_workers
    n_chunks = per_w // C
    assert per_w % C == 0 and Bb % per_w == 0
    wpt = Bb // per_w

    mesh = plsc.VectorSubcoreMesh(core_axis_name="c", subcore_axis_name="s")

    @functools.partial(
        pl.kernel,
        mesh=mesh,
        out_type=jax.ShapeDtypeStruct((T, Bb, Ep), jnp.float32),
        scratch_types=[
            pltpu.VMEM((per_w,), jnp.int32),
            pltpu.VMEM((2, C, Ep), jnp.float32),
            pltpu.SemaphoreType.DMA((2,)),
            pltpu.SemaphoreType.DMA((2,)),
        ],
    )
    def gather_kernel(table_hbm, idx_hbm, out_hbm, idx_v, rows_v, gsem, wsem):
        wid = lax.axis_index("s") * info.num_cores + lax.axis_index("c")
        base = wid * per_w
        t_id = wid // wpt
        b_base = (wid % wpt) * per_w
        pltpu.sync_copy(idx_hbm.at[pl.ds(base, per_w)], idx_v)

        def body(i, carry):
            s = lax.rem(i, 2)

            @pl.when(i >= 2)
            def _():
                pltpu.make_async_copy(
                    rows_v.at[s],
                    out_hbm.at[t_id, pl.ds(b_base + (i - 2) * C, C), :],
                    wsem.at[s],
                ).wait()

            pltpu.async_copy(
                table_hbm.at[idx_v.at[pl.ds(i * C, C)]], rows_v.at[s], gsem.at[s]
            ).wait()
            pltpu.make_async_copy(
                rows_v.at[s],
                out_hbm.at[t_id, pl.ds(b_base + i * C, C), :],
                wsem.at[s],
            ).start()
            return carry

        lax.fori_loop(0, n_chunks, body, 0)
        for i in range(max(n_chunks - 2, 0), n_chunks):
            s = i % 2
            pltpu.make_async_copy(
                rows_v.at[s],
                out_hbm.at[t_id, pl.ds(b_base + i * C, C), :],
                wsem.at[s],
            ).wait()

    return gather_kernel


def _head_body(emb_ref, wt_ref, b_ref, out_ref):
    _, vc, bb = out_ref.shape
    acc = (
        lax.dot_general(
            wt_ref[...],
            emb_ref[0],
            dimension_numbers=(((1,), (1,)), ((), ())),
            preferred_element_type=jnp.float32,
        )
        + b_ref[...]
    )
    out_ref[...] = acc.reshape(1, vc, bb)


@functools.cache
def _make_head_t(T, Ep, Bb, Vo, VC, BC):
    grid = (T, Bb // BC, Vo // VC)
    return pl.pallas_call(
        _head_body,
        grid=grid,
        in_specs=[
            pl.BlockSpec((1, BC, Ep), lambda t, k, j: (t, k, 0)),
            pl.BlockSpec((VC, Ep), lambda t, k, j: (j, 0)),
            pl.BlockSpec((VC, 1), lambda t, k, j: (j, 0)),
        ],
        out_specs=pl.BlockSpec((1, VC, BC), lambda t, k, j: (t, j, k)),
        out_shape=jax.ShapeDtypeStruct((T, Vo, Bb), jnp.float32),
    )


def kernel(idx, tok_table, pos_table, W, b):
    Bb, T = idx.shape
    V, E = tok_table.shape
    Vo = W.shape[1]
    Ep = 128

    tok_p = jnp.pad(tok_table, ((0, 0), (0, Ep - E)))
    Wt_p = jnp.pad(W.T, ((0, 0), (0, Ep - E)))
    bcol = b.reshape(Vo, 1)
    idx_t = idx.T.reshape(-1).astype(jnp.int32)

    emb3 = _make_gather(V, Ep, T, Bb, 256)(tok_p, idx_t)
    out_tr = _make_head_t(T, Ep, Bb, Vo, 200, 8192)(emb3, Wt_p, bcol)
    return jnp.transpose(out_tr, (2, 0, 1))


# speedup vs baseline: 2.8373x; 1.0221x over previous
import functools

import jax
import jax.numpy as jnp
from jax import lax
from jax.experimental import pallas as pl
from jax.experimental.pallas import tpu as pltpu
from jax.experimental.pallas import tpu_sc as plsc


@functools.cache
def _make_gather(V, Ep, T, Bb, C):
    info = plsc.get_sparse_core_info()
    num_workers = info.num_cores * info.num_subcores
    per_w = T * Bb // num_workers
    n_chunks = per_w // C
    assert per_w % C == 0 and Bb % per_w == 0
    wpt = Bb // per_w

    mesh = plsc.VectorSubcoreMesh(core_axis_name="c", subcore_axis_name="s")

    @functools.partial(
        pl.kernel,
        mesh=mesh,
        out_type=jax.ShapeDtypeStruct((T, Bb, Ep), jnp.float32),
        scratch_types=[
            pltpu.VMEM((per_w,), jnp.int32),
            pltpu.VMEM((2, C, Ep), jnp.float32),
            pltpu.SemaphoreType.DMA((2,)),
            pltpu.SemaphoreType.DMA((2,)),
        ],
    )
    def gather_kernel(table_hbm, idx_hbm, out_hbm, idx_v, rows_v, gsem, wsem):
        wid = lax.axis_index("s") * info.num_cores + lax.axis_index("c")
        base = wid * per_w
        t_id = wid // wpt
        b_base = (wid % wpt) * per_w
        pltpu.sync_copy(idx_hbm.at[pl.ds(base, per_w)], idx_v)

        def body(i, carry):
            s = lax.rem(i, 2)

            @pl.when(i >= 2)
            def _():
                pltpu.make_async_copy(
                    rows_v.at[s],
                    out_hbm.at[t_id, pl.ds(b_base + (i - 2) * C, C), :],
                    wsem.at[s],
                ).wait()

            pltpu.async_copy(
                table_hbm.at[idx_v.at[pl.ds(i * C, C)]], rows_v.at[s], gsem.at[s]
            ).wait()
            pltpu.make_async_copy(
                rows_v.at[s],
                out_hbm.at[t_id, pl.ds(b_base + i * C, C), :],
                wsem.at[s],
            ).start()
            return carry

        lax.fori_loop(0, n_chunks, body, 0)
        for i in range(max(n_chunks - 2, 0), n_chunks):
            s = i % 2
            pltpu.make_async_copy(
                rows_v.at[s],
                out_hbm.at[t_id, pl.ds(b_base + i * C, C), :],
                wsem.at[s],
            ).wait()

    return gather_kernel


def _head_body(emb_ref, wt_ref, b_ref, out_ref):
    _, vc, bb = out_ref.shape
    acc = (
        lax.dot_general(
            wt_ref[...],
            emb_ref[0],
            dimension_numbers=(((1,), (1,)), ((), ())),
            preferred_element_type=jnp.float32,
        )
        + b_ref[...]
    )
    out_ref[...] = acc.reshape(1, vc, bb)


def _head_body_alias(emb_ref, wt_ref, b_ref, prev_ref, out_ref):
    _head_body(emb_ref, wt_ref, b_ref, out_ref)


@functools.cache
def _make_head_t(T, Tg, s0, Ep, Bb, Vo, VC, BC, alias):
    grid = (Tg, Bb // BC, Vo // VC)
    in_specs = [
        pl.BlockSpec((1, BC, Ep), lambda t, k, j: (t, k, 0)),
        pl.BlockSpec((VC, Ep), lambda t, k, j: (j, 0)),
        pl.BlockSpec((VC, 1), lambda t, k, j: (j, 0)),
    ]
    kwargs = {}
    if alias:
        in_specs.append(pl.BlockSpec(memory_space=pl.ANY))
        kwargs["input_output_aliases"] = {3: 0}
    return pl.pallas_call(
        _head_body_alias if alias else _head_body,
        grid=grid,
        in_specs=in_specs,
        out_specs=pl.BlockSpec((1, VC, BC), lambda t, k, j: (t + s0, j, k)),
        out_shape=jax.ShapeDtypeStruct((T, Vo, Bb), jnp.float32),
        **kwargs,
    )


def kernel(idx, tok_table, pos_table, W, b):
    Bb, T = idx.shape
    V, E = tok_table.shape
    Vo = W.shape[1]
    Ep = 128
    NS = 4
    Tg = T // NS

    tok_p = jnp.pad(tok_table, ((0, 0), (0, Ep - E)))
    Wt_p = jnp.pad(W.T, ((0, 0), (0, Ep - E)))
    bcol = b.reshape(Vo, 1)
    idx_t = idx.T.reshape(-1).astype(jnp.int32)

    gather = _make_gather(V, Ep, Tg, Bb, 256)
    embs = [
        gather(tok_p, lax.slice(idx_t, (s * Tg * Bb,), ((s + 1) * Tg * Bb,)))
        for s in range(NS)
    ]
    out_tr = _make_head_t(T, Tg, 0, Ep, Bb, Vo, 200, 8192, False)(
        embs[0], Wt_p, bcol
    )
    for s in range(1, NS):
        out_tr = _make_head_t(T, Tg, s * Tg, Ep, Bb, Vo, 200, 8192, True)(
            embs[s], Wt_p, bcol, out_tr
        )
    return jnp.transpose(out_tr, (2, 0, 1))
